# Initial kernel scaffold; baseline (speedup 1.0000x reference)
import jax
import jax.numpy as jnp
from jax.experimental import pallas as pl


def kernel(x, edge_index, batch, We1, be1, We2, be2, Wc0a, bc0a, Wc0b, bc0b, Wc1a, bc1a, Wc1b, bc1b, Wl1, bl1, Wl2, bl2):
    raise NotImplementedError("write your pallas kernel here")


# speedup vs baseline: 3.5405x; 3.5405x over previous
import functools

import jax
import jax.numpy as jnp
from jax import lax
from jax.experimental import pallas as pl
from jax.experimental.pallas import tpu as pltpu
from jax.experimental.pallas import tpu_sc as plsc

N = 10000
E = 320000
D_IN = 128
D_H = 256
HALF = 128
N_CLASSES = 10
N_GRAPHS = 64

NC = 2
NS = 16
EPW = E // NS
K = 80
NCHUNK = EPW // K
RPT = 632
RPT_LAST = N - (NS - 1) * RPT

BR = 2000
NBLK = N // BR


def _rowwise_copy(c, s, srcs, dsts):
    base = pl.multiple_of(s * RPT, 8)
    for cid in (0, 1):
        for sz, b in ((RPT, base), (RPT_LAST, (NS - 1) * RPT)):
            is_last = sz == RPT_LAST

            @pl.when((c == cid) & ((s == NS - 1) == is_last))
            def _(cid=cid, sz=sz, b=b):
                pltpu.sync_copy(srcs[cid].at[pl.ds(b, sz)],
                                dsts[cid].at[pl.ds(b, sz)])


def _sc_segsum_body(hA, hB, src, dst, zA, zB, acc, sidx, didx, rows, sem):
    c = lax.axis_index("c")
    s = lax.axis_index("s")

    _rowwise_copy(c, s, (hA, hB), (acc, acc))
    plsc.subcore_barrier()

    def chunk(i, carry):
        base = pl.multiple_of(s * EPW + i * K, 8)
        pltpu.sync_copy(src.at[pl.ds(base, K)], sidx)
        pltpu.sync_copy(dst.at[pl.ds(base, K)], didx)

        @pl.when(c == 0)
        def _():
            pltpu.async_copy(hA.at[sidx], rows, sem).wait()

        @pl.when(c == 1)
        def _():
            pltpu.async_copy(hB.at[sidx], rows, sem).wait()

        pltpu.sync_copy(rows, acc.at[didx], add=True)
        return carry

    lax.fori_loop(0, NCHUNK, chunk, 0)
    plsc.subcore_barrier()

    _rowwise_copy(c, s, (acc, acc), (zA, zB))


@functools.cache
def _sc_segsum_call():
    return pl.kernel(
        _sc_segsum_body,
        out_type=(
            jax.ShapeDtypeStruct((N, HALF), jnp.float32),
            jax.ShapeDtypeStruct((N, HALF), jnp.float32),
        ),
        mesh=plsc.VectorSubcoreMesh(core_axis_name="c", subcore_axis_name="s"),
        scratch_types=[
            pltpu.VMEM_SHARED((N, HALF), jnp.float32),
            pltpu.VMEM((K,), jnp.int32),
            pltpu.VMEM((K,), jnp.int32),
            pltpu.VMEM((K, HALF), jnp.float32),
            pltpu.SemaphoreType.DMA,
        ],
    )


def _sc_segsum(hA, hB, src, dst):
    return _sc_segsum_call()(hA, hB, src, dst)


def _enc_body(x_ref, w1, b1, w2, b2, oa, ob):
    h = jnp.maximum(jnp.dot(x_ref[...], w1[...],
                            preferred_element_type=jnp.float32) + b1[...], 0.0)
    h = jnp.maximum(jnp.dot(h, w2[...],
                            preferred_element_type=jnp.float32) + b2[...], 0.0)
    oa[...] = h[:, :HALF]
    ob[...] = h[:, HALF:]


def _enc(x, W1, b1, W2, b2):
    return pl.pallas_call(
        _enc_body,
        grid=(NBLK,),
        in_specs=[
            pl.BlockSpec((BR, D_IN), lambda i: (i, 0)),
            pl.BlockSpec((D_IN, D_H), lambda i: (0, 0)),
            pl.BlockSpec((D_H,), lambda i: (0,)),
            pl.BlockSpec((D_H, D_H), lambda i: (0, 0)),
            pl.BlockSpec((D_H,), lambda i: (0,)),
        ],
        out_specs=[
            pl.BlockSpec((BR, HALF), lambda i: (i, 0)),
            pl.BlockSpec((BR, HALF), lambda i: (i, 0)),
        ],
        out_shape=[
            jax.ShapeDtypeStruct((N, HALF), jnp.float32),
            jax.ShapeDtypeStruct((N, HALF), jnp.float32),
        ],
    )(x, W1, b1, W2, b2)


def _mlp_body(za, zb, w1, b1, w2, b2, oa, ob):
    w1v = w1[...]
    t = (jnp.dot(za[...], w1v[:HALF], preferred_element_type=jnp.float32)
         + jnp.dot(zb[...], w1v[HALF:], preferred_element_type=jnp.float32)
         + b1[...])
    t = jnp.maximum(t, 0.0)
    o = jnp.dot(t, w2[...], preferred_element_type=jnp.float32) + b2[...]
    oa[...] = o[:, :HALF]
    ob[...] = o[:, HALF:]


def _mlp(za, zb, W1, b1, W2, b2):
    return pl.pallas_call(
        _mlp_body,
        grid=(NBLK,),
        in_specs=[
            pl.BlockSpec((BR, HALF), lambda i: (i, 0)),
            pl.BlockSpec((BR, HALF), lambda i: (i, 0)),
            pl.BlockSpec((D_H, D_H), lambda i: (0, 0)),
            pl.BlockSpec((D_H,), lambda i: (0,)),
            pl.BlockSpec((D_H, D_H), lambda i: (0, 0)),
            pl.BlockSpec((D_H,), lambda i: (0,)),
        ],
        out_specs=[
            pl.BlockSpec((BR, HALF), lambda i: (i, 0)),
            pl.BlockSpec((BR, HALF), lambda i: (i, 0)),
        ],
        out_shape=[
            jax.ShapeDtypeStruct((N, HALF), jnp.float32),
            jax.ShapeDtypeStruct((N, HALF), jnp.float32),
        ],
    )(za, zb, W1, b1, W2, b2)


def _final_body(za, zb, w1, b1, w2, b2, batch_ref, wl1, bl1, wl2, bl2,
                out_ref, pooled):
    i = pl.program_id(0)
    w1v = w1[...]
    t = (jnp.dot(za[...], w1v[:HALF], preferred_element_type=jnp.float32)
         + jnp.dot(zb[...], w1v[HALF:], preferred_element_type=jnp.float32)
         + b1[...])
    t = jnp.maximum(t, 0.0)
    h3 = jnp.dot(t, w2[...], preferred_element_type=jnp.float32) + b2[...]

    b_ids = batch_ref[0, 0, :]
    gids = lax.broadcasted_iota(jnp.int32, (N_GRAPHS, BR), 0)
    onehot = (b_ids[None, :] == gids).astype(jnp.float32)
    part = jnp.dot(onehot, h3, preferred_element_type=jnp.float32)

    @pl.when(i == 0)
    def _():
        pooled[...] = part

    @pl.when(i > 0)
    def _():
        pooled[...] = pooled[...] + part

    @pl.when(i == NBLK - 1)
    def _():
        p = jnp.maximum(jnp.dot(pooled[...], wl1[...],
                                preferred_element_type=jnp.float32) + bl1[...],
                        0.0)
        out_ref[...] = jnp.dot(p, wl2[...],
                               preferred_element_type=jnp.float32) + bl2[...]


def _final(za, zb, W1, b1, W2, b2, batch3, Wl1, bl1, Wl2, bl2):
    return pl.pallas_call(
        _final_body,
        grid=(NBLK,),
        in_specs=[
            pl.BlockSpec((BR, HALF), lambda i: (i, 0)),
            pl.BlockSpec((BR, HALF), lambda i: (i, 0)),
            pl.BlockSpec((D_H, D_H), lambda i: (0, 0)),
            pl.BlockSpec((D_H,), lambda i: (0,)),
            pl.BlockSpec((D_H, D_H), lambda i: (0, 0)),
            pl.BlockSpec((D_H,), lambda i: (0,)),
            pl.BlockSpec((1, 1, BR), lambda i: (i, 0, 0)),
            pl.BlockSpec((D_H, D_H), lambda i: (0, 0)),
            pl.BlockSpec((D_H,), lambda i: (0,)),
            pl.BlockSpec((D_H, N_CLASSES), lambda i: (0, 0)),
            pl.BlockSpec((N_CLASSES,), lambda i: (0,)),
        ],
        out_specs=pl.BlockSpec((N_GRAPHS, N_CLASSES), lambda i: (0, 0)),
        out_shape=jax.ShapeDtypeStruct((N_GRAPHS, N_CLASSES), jnp.float32),
        scratch_shapes=[pltpu.VMEM((N_GRAPHS, D_H), jnp.float32)],
    )(za, zb, W1, b1, W2, b2, batch3, Wl1, bl1, Wl2, bl2)


def kernel(x, edge_index, batch, We1, be1, We2, be2, Wc0a, bc0a, Wc0b, bc0b,
           Wc1a, bc1a, Wc1b, bc1b, Wl1, bl1, Wl2, bl2):
    src = edge_index[0].astype(jnp.int32)
    dst = edge_index[1].astype(jnp.int32)
    batch3 = batch.astype(jnp.int32).reshape(NBLK, 1, BR)

    hA, hB = _enc(x, We1, be1, We2, be2)
    zA, zB = _sc_segsum(hA, hB, src, dst)
    h2A, h2B = _mlp(zA, zB, Wc0a, bc0a, Wc0b, bc0b)
    z2A, z2B = _sc_segsum(h2A, h2B, src, dst)
    return _final(z2A, z2B, Wc1a, bc1a, Wc1b, bc1b, batch3, Wl1, bl1, Wl2, bl2)


# speedup vs baseline: 10.8832x; 3.0739x over previous
import functools

import jax
import jax.numpy as jnp
from jax import lax
from jax.experimental import pallas as pl
from jax.experimental.pallas import tpu as pltpu
from jax.experimental.pallas import tpu_sc as plsc

N = 10000
E = 320000
D_IN = 128
D_H = 256
HALF = 128
N_CLASSES = 10
N_GRAPHS = 64

NC = 2
NS = 16
EPW = E // NS
K = 40
NCHUNK = EPW // K
NBUF = 5
GD = 3
NI = 10
IDL = 8
RPT = 632
RPT_LAST = N - (NS - 1) * RPT

BR = 2000
NBLK = N // BR


def _rowwise_copy(c, s, srcs, dsts):
    base = pl.multiple_of(s * RPT, 8)
    for cid in (0, 1):
        for sz, b in ((RPT, base), (RPT_LAST, (NS - 1) * RPT)):
            is_last = sz == RPT_LAST

            @pl.when((c == cid) & ((s == NS - 1) == is_last))
            def _(cid=cid, sz=sz, b=b):
                pltpu.sync_copy(srcs[cid].at[pl.ds(b, sz)],
                                dsts[cid].at[pl.ds(b, sz)])


def _sc_segsum_body(hA, hB, src, dst, zA, zB, acc, *bufs):
    rows = bufs[:NBUF]
    sidx = bufs[NBUF:NBUF + NI]
    didx = bufs[NBUF + NI:NBUF + 2 * NI]
    gsem = bufs[NBUF + 2 * NI:2 * NBUF + 2 * NI]
    ssem = bufs[2 * NBUF + 2 * NI:3 * NBUF + 2 * NI]
    isem = bufs[3 * NBUF + 2 * NI:3 * NBUF + 3 * NI]
    c = lax.axis_index("c")
    s = lax.axis_index("s")

    def i_descs(jc, bi):
        off = pl.multiple_of(s * EPW + jc * K, 8)
        return (pltpu.make_async_copy(src.at[pl.ds(off, K)], sidx[bi],
                                      isem[bi]),
                pltpu.make_async_copy(dst.at[pl.ds(off, K)], didx[bi],
                                      isem[bi]))

    def g_desc(table, jc, b, bi):
        return pltpu.make_async_copy(table.at[sidx[bi]], rows[b], gsem[b])

    def s_desc(jc, b, bi):
        return pltpu.make_async_copy(rows[b], acc.at[didx[bi]], ssem[b])

    def issue_gather(jc, b, bi):
        for d in i_descs(jc, bi):
            d.wait()

        @pl.when(c == 0)
        def _():
            g_desc(hA, jc, b, bi).start()

        @pl.when(c == 1)
        def _():
            g_desc(hB, jc, b, bi).start()

    for bi in range(IDL):
        for d in i_descs(bi, bi):
            d.start()
    _rowwise_copy(c, s, (hA, hB), (acc, acc))
    for b in range(GD):
        issue_gather(b, b, b)

    plsc.subcore_barrier()

    @pl.loop(0, NCHUNK, step=NI)
    def _(j):
        for b0 in range(NI):
            jc = j + b0
            b = b0 % NBUF
            b2 = (b0 + GD) % NBUF
            bi2 = (b0 + GD) % NI
            bi3 = (b0 + IDL) % NI

            @pl.when(jc + GD < NCHUNK)
            def _():
                @pl.when(jc + GD >= NBUF)
                def _():
                    s_desc(jc + GD - NBUF, b2, (b0 + GD - NBUF) % NI).wait()

                issue_gather(jc + GD, b2, bi2)

            g_desc(hA, jc, b, b0).wait()
            s_desc(jc, b, b0).start(add=True)

            @pl.when(jc + IDL < NCHUNK)
            def _():
                for d in i_descs(jc + IDL, bi3):
                    d.start()

    for b0 in range(NBUF):
        jc_last = NCHUNK - NBUF + b0
        s_desc(jc_last, jc_last % NBUF, jc_last % NI).wait()

    plsc.subcore_barrier()

    _rowwise_copy(c, s, (acc, acc), (zA, zB))


@functools.cache
def _sc_segsum_call():
    return pl.kernel(
        _sc_segsum_body,
        out_type=(
            jax.ShapeDtypeStruct((N, HALF), jnp.float32),
            jax.ShapeDtypeStruct((N, HALF), jnp.float32),
        ),
        mesh=plsc.VectorSubcoreMesh(core_axis_name="c", subcore_axis_name="s"),
        scratch_types=(
            [pltpu.VMEM_SHARED((N, HALF), jnp.float32)]
            + [pltpu.VMEM((K, HALF), jnp.float32) for _ in range(NBUF)]
            + [pltpu.VMEM((K,), jnp.int32) for _ in range(2 * NI)]
            + [pltpu.SemaphoreType.DMA for _ in range(2 * NBUF + NI)]
        ),
    )


def _sc_segsum(hA, hB, src, dst):
    return _sc_segsum_call()(hA, hB, src, dst)


def _enc_body(x_ref, w1, b1, w2, b2, oa, ob):
    h = jnp.maximum(jnp.dot(x_ref[...], w1[...],
                            preferred_element_type=jnp.float32) + b1[...], 0.0)
    h = jnp.maximum(jnp.dot(h, w2[...],
                            preferred_element_type=jnp.float32) + b2[...], 0.0)
    oa[...] = h[:, :HALF]
    ob[...] = h[:, HALF:]


def _enc(x, W1, b1, W2, b2):
    return pl.pallas_call(
        _enc_body,
        grid=(NBLK,),
        in_specs=[
            pl.BlockSpec((BR, D_IN), lambda i: (i, 0)),
            pl.BlockSpec((D_IN, D_H), lambda i: (0, 0)),
            pl.BlockSpec((D_H,), lambda i: (0,)),
            pl.BlockSpec((D_H, D_H), lambda i: (0, 0)),
            pl.BlockSpec((D_H,), lambda i: (0,)),
        ],
        out_specs=[
            pl.BlockSpec((BR, HALF), lambda i: (i, 0)),
            pl.BlockSpec((BR, HALF), lambda i: (i, 0)),
        ],
        out_shape=[
            jax.ShapeDtypeStruct((N, HALF), jnp.float32),
            jax.ShapeDtypeStruct((N, HALF), jnp.float32),
        ],
    )(x, W1, b1, W2, b2)


def _mlp_body(za, zb, w1, b1, w2, b2, oa, ob):
    w1v = w1[...]
    t = (jnp.dot(za[...], w1v[:HALF], preferred_element_type=jnp.float32)
         + jnp.dot(zb[...], w1v[HALF:], preferred_element_type=jnp.float32)
         + b1[...])
    t = jnp.maximum(t, 0.0)
    o = jnp.dot(t, w2[...], preferred_element_type=jnp.float32) + b2[...]
    oa[...] = o[:, :HALF]
    ob[...] = o[:, HALF:]


def _mlp(za, zb, W1, b1, W2, b2):
    return pl.pallas_call(
        _mlp_body,
        grid=(NBLK,),
        in_specs=[
            pl.BlockSpec((BR, HALF), lambda i: (i, 0)),
            pl.BlockSpec((BR, HALF), lambda i: (i, 0)),
            pl.BlockSpec((D_H, D_H), lambda i: (0, 0)),
            pl.BlockSpec((D_H,), lambda i: (0,)),
            pl.BlockSpec((D_H, D_H), lambda i: (0, 0)),
            pl.BlockSpec((D_H,), lambda i: (0,)),
        ],
        out_specs=[
            pl.BlockSpec((BR, HALF), lambda i: (i, 0)),
            pl.BlockSpec((BR, HALF), lambda i: (i, 0)),
        ],
        out_shape=[
            jax.ShapeDtypeStruct((N, HALF), jnp.float32),
            jax.ShapeDtypeStruct((N, HALF), jnp.float32),
        ],
    )(za, zb, W1, b1, W2, b2)


def _final_body(za, zb, w1, b1, w2, b2, batch_ref, wl1, bl1, wl2, bl2,
                out_ref, pooled):
    i = pl.program_id(0)
    w1v = w1[...]
    t = (jnp.dot(za[...], w1v[:HALF], preferred_element_type=jnp.float32)
         + jnp.dot(zb[...], w1v[HALF:], preferred_element_type=jnp.float32)
         + b1[...])
    t = jnp.maximum(t, 0.0)
    h3 = jnp.dot(t, w2[...], preferred_element_type=jnp.float32) + b2[...]

    b_ids = batch_ref[0, 0, :]
    gids = lax.broadcasted_iota(jnp.int32, (N_GRAPHS, BR), 0)
    onehot = (b_ids[None, :] == gids).astype(jnp.float32)
    part = jnp.dot(onehot, h3, preferred_element_type=jnp.float32)

    @pl.when(i == 0)
    def _():
        pooled[...] = part

    @pl.when(i > 0)
    def _():
        pooled[...] = pooled[...] + part

    @pl.when(i == NBLK - 1)
    def _():
        p = jnp.maximum(jnp.dot(pooled[...], wl1[...],
                                preferred_element_type=jnp.float32) + bl1[...],
                        0.0)
        out_ref[...] = jnp.dot(p, wl2[...],
                               preferred_element_type=jnp.float32) + bl2[...]


def _final(za, zb, W1, b1, W2, b2, batch3, Wl1, bl1, Wl2, bl2):
    return pl.pallas_call(
        _final_body,
        grid=(NBLK,),
        in_specs=[
            pl.BlockSpec((BR, HALF), lambda i: (i, 0)),
            pl.BlockSpec((BR, HALF), lambda i: (i, 0)),
            pl.BlockSpec((D_H, D_H), lambda i: (0, 0)),
            pl.BlockSpec((D_H,), lambda i: (0,)),
            pl.BlockSpec((D_H, D_H), lambda i: (0, 0)),
            pl.BlockSpec((D_H,), lambda i: (0,)),
            pl.BlockSpec((1, 1, BR), lambda i: (i, 0, 0)),
            pl.BlockSpec((D_H, D_H), lambda i: (0, 0)),
            pl.BlockSpec((D_H,), lambda i: (0,)),
            pl.BlockSpec((D_H, N_CLASSES), lambda i: (0, 0)),
            pl.BlockSpec((N_CLASSES,), lambda i: (0,)),
        ],
        out_specs=pl.BlockSpec((N_GRAPHS, N_CLASSES), lambda i: (0, 0)),
        out_shape=jax.ShapeDtypeStruct((N_GRAPHS, N_CLASSES), jnp.float32),
        scratch_shapes=[pltpu.VMEM((N_GRAPHS, D_H), jnp.float32)],
    )(za, zb, W1, b1, W2, b2, batch3, Wl1, bl1, Wl2, bl2)


def kernel(x, edge_index, batch, We1, be1, We2, be2, Wc0a, bc0a, Wc0b, bc0b,
           Wc1a, bc1a, Wc1b, bc1b, Wl1, bl1, Wl2, bl2):
    src = edge_index[0].astype(jnp.int32)
    dst = edge_index[1].astype(jnp.int32)
    batch3 = batch.astype(jnp.int32).reshape(NBLK, 1, BR)

    hA, hB = _enc(x, We1, be1, We2, be2)
    zA, zB = _sc_segsum(hA, hB, src, dst)
    h2A, h2B = _mlp(zA, zB, Wc0a, bc0a, Wc0b, bc0b)
    z2A, z2B = _sc_segsum(h2A, h2B, src, dst)
    return _final(z2A, z2B, Wc1a, bc1a, Wc1b, bc1b, batch3, Wl1, bl1, Wl2, bl2)


# speedup vs baseline: 11.3833x; 1.0460x over previous
import functools

import jax
import jax.numpy as jnp
from jax import lax
from jax.experimental import pallas as pl
from jax.experimental.pallas import tpu as pltpu
from jax.experimental.pallas import tpu_sc as plsc

N = 10000
E = 320000
D_IN = 128
D_H = 256
HALF = 128
N_CLASSES = 10
N_GRAPHS = 64

NC = 2
NS = 16
EPW = E // NS
K = 40
NCHUNK = EPW // K
NBUF = 5
GD = 4
NI = 10
IDL = 8
RPT = 632
RPT_LAST = N - (NS - 1) * RPT

BR = 2000
NBLK = N // BR


def _rowwise_copy(c, s, srcs, dsts):
    base = pl.multiple_of(s * RPT, 8)
    for cid in (0, 1):
        for sz, b in ((RPT, base), (RPT_LAST, (NS - 1) * RPT)):
            is_last = sz == RPT_LAST

            @pl.when((c == cid) & ((s == NS - 1) == is_last))
            def _(cid=cid, sz=sz, b=b):
                pltpu.sync_copy(srcs[cid].at[pl.ds(b, sz)],
                                dsts[cid].at[pl.ds(b, sz)])


def _sc_segsum_body(hA, hB, src, dst, zA, zB, acc, *bufs):
    rows = bufs[:NBUF]
    sidx = bufs[NBUF:NBUF + NI]
    didx = bufs[NBUF + NI:NBUF + 2 * NI]
    gsem = bufs[NBUF + 2 * NI:2 * NBUF + 2 * NI]
    ssem = bufs[2 * NBUF + 2 * NI:3 * NBUF + 2 * NI]
    isem = bufs[3 * NBUF + 2 * NI:3 * NBUF + 3 * NI]
    c = lax.axis_index("c")
    s = lax.axis_index("s")

    def i_descs(jc, bi):
        off = pl.multiple_of(s * EPW + jc * K, 8)
        return (pltpu.make_async_copy(src.at[pl.ds(off, K)], sidx[bi],
                                      isem[bi]),
                pltpu.make_async_copy(dst.at[pl.ds(off, K)], didx[bi],
                                      isem[bi]))

    def g_desc(table, jc, b, bi):
        return pltpu.make_async_copy(table.at[sidx[bi]], rows[b], gsem[b])

    def s_desc(jc, b, bi):
        return pltpu.make_async_copy(rows[b], acc.at[didx[bi]], ssem[b])

    def issue_gather(jc, b, bi):
        for d in i_descs(jc, bi):
            d.wait()

        @pl.when(c == 0)
        def _():
            g_desc(hA, jc, b, bi).start()

        @pl.when(c == 1)
        def _():
            g_desc(hB, jc, b, bi).start()

    for bi in range(IDL):
        for d in i_descs(bi, bi):
            d.start()
    _rowwise_copy(c, s, (hA, hB), (acc, acc))
    for b in range(GD):
        issue_gather(b, b, b)

    plsc.subcore_barrier()

    @pl.loop(0, NCHUNK, step=NI)
    def _(j):
        for b0 in range(NI):
            jc = j + b0
            b = b0 % NBUF
            b2 = (b0 + GD) % NBUF
            bi2 = (b0 + GD) % NI
            bi3 = (b0 + IDL) % NI

            @pl.when(jc + GD < NCHUNK)
            def _():
                @pl.when(jc + GD >= NBUF)
                def _():
                    s_desc(jc + GD - NBUF, b2, (b0 + GD - NBUF) % NI).wait()

                issue_gather(jc + GD, b2, bi2)

            g_desc(hA, jc, b, b0).wait()
            s_desc(jc, b, b0).start(add=True)

            @pl.when(jc + IDL < NCHUNK)
            def _():
                for d in i_descs(jc + IDL, bi3):
                    d.start()

    for b0 in range(NBUF):
        jc_last = NCHUNK - NBUF + b0
        s_desc(jc_last, jc_last % NBUF, jc_last % NI).wait()

    plsc.subcore_barrier()

    _rowwise_copy(c, s, (acc, acc), (zA, zB))


@functools.cache
def _sc_segsum_call():
    return pl.kernel(
        _sc_segsum_body,
        out_type=(
            jax.ShapeDtypeStruct((N, HALF), jnp.float32),
            jax.ShapeDtypeStruct((N, HALF), jnp.float32),
        ),
        mesh=plsc.VectorSubcoreMesh(core_axis_name="c", subcore_axis_name="s"),
        scratch_types=(
            [pltpu.VMEM_SHARED((N, HALF), jnp.float32)]
            + [pltpu.VMEM((K, HALF), jnp.float32) for _ in range(NBUF)]
            + [pltpu.VMEM((K,), jnp.int32) for _ in range(2 * NI)]
            + [pltpu.SemaphoreType.DMA for _ in range(2 * NBUF + NI)]
        ),
    )


def _sc_segsum(hA, hB, src, dst):
    return _sc_segsum_call()(hA, hB, src, dst)


def _enc_body(x_ref, w1, b1, w2, b2, oa, ob):
    h = jnp.maximum(jnp.dot(x_ref[...], w1[...],
                            preferred_element_type=jnp.float32) + b1[...], 0.0)
    h = jnp.maximum(jnp.dot(h, w2[...],
                            preferred_element_type=jnp.float32) + b2[...], 0.0)
    oa[...] = h[:, :HALF]
    ob[...] = h[:, HALF:]


def _enc(x, W1, b1, W2, b2):
    return pl.pallas_call(
        _enc_body,
        grid=(NBLK,),
        in_specs=[
            pl.BlockSpec((BR, D_IN), lambda i: (i, 0)),
            pl.BlockSpec((D_IN, D_H), lambda i: (0, 0)),
            pl.BlockSpec((D_H,), lambda i: (0,)),
            pl.BlockSpec((D_H, D_H), lambda i: (0, 0)),
            pl.BlockSpec((D_H,), lambda i: (0,)),
        ],
        out_specs=[
            pl.BlockSpec((BR, HALF), lambda i: (i, 0)),
            pl.BlockSpec((BR, HALF), lambda i: (i, 0)),
        ],
        out_shape=[
            jax.ShapeDtypeStruct((N, HALF), jnp.float32),
            jax.ShapeDtypeStruct((N, HALF), jnp.float32),
        ],
    )(x, W1, b1, W2, b2)


def _mlp_body(za, zb, w1, b1, w2, b2, oa, ob):
    w1v = w1[...]
    t = (jnp.dot(za[...], w1v[:HALF], preferred_element_type=jnp.float32)
         + jnp.dot(zb[...], w1v[HALF:], preferred_element_type=jnp.float32)
         + b1[...])
    t = jnp.maximum(t, 0.0)
    o = jnp.dot(t, w2[...], preferred_element_type=jnp.float32) + b2[...]
    oa[...] = o[:, :HALF]
    ob[...] = o[:, HALF:]


def _mlp(za, zb, W1, b1, W2, b2):
    return pl.pallas_call(
        _mlp_body,
        grid=(NBLK,),
        in_specs=[
            pl.BlockSpec((BR, HALF), lambda i: (i, 0)),
            pl.BlockSpec((BR, HALF), lambda i: (i, 0)),
            pl.BlockSpec((D_H, D_H), lambda i: (0, 0)),
            pl.BlockSpec((D_H,), lambda i: (0,)),
            pl.BlockSpec((D_H, D_H), lambda i: (0, 0)),
            pl.BlockSpec((D_H,), lambda i: (0,)),
        ],
        out_specs=[
            pl.BlockSpec((BR, HALF), lambda i: (i, 0)),
            pl.BlockSpec((BR, HALF), lambda i: (i, 0)),
        ],
        out_shape=[
            jax.ShapeDtypeStruct((N, HALF), jnp.float32),
            jax.ShapeDtypeStruct((N, HALF), jnp.float32),
        ],
    )(za, zb, W1, b1, W2, b2)


def _final_body(za, zb, w1, b1, w2, b2, batch_ref, wl1, bl1, wl2, bl2,
                out_ref, pooled):
    i = pl.program_id(0)
    w1v = w1[...]
    t = (jnp.dot(za[...], w1v[:HALF], preferred_element_type=jnp.float32)
         + jnp.dot(zb[...], w1v[HALF:], preferred_element_type=jnp.float32)
         + b1[...])
    t = jnp.maximum(t, 0.0)
    h3 = jnp.dot(t, w2[...], preferred_element_type=jnp.float32) + b2[...]

    b_ids = batch_ref[0, 0, :]
    gids = lax.broadcasted_iota(jnp.int32, (N_GRAPHS, BR), 0)
    onehot = (b_ids[None, :] == gids).astype(jnp.float32)
    part = jnp.dot(onehot, h3, preferred_element_type=jnp.float32)

    @pl.when(i == 0)
    def _():
        pooled[...] = part

    @pl.when(i > 0)
    def _():
        pooled[...] = pooled[...] + part

    @pl.when(i == NBLK - 1)
    def _():
        p = jnp.maximum(jnp.dot(pooled[...], wl1[...],
                                preferred_element_type=jnp.float32) + bl1[...],
                        0.0)
        out_ref[...] = jnp.dot(p, wl2[...],
                               preferred_element_type=jnp.float32) + bl2[...]


def _final(za, zb, W1, b1, W2, b2, batch3, Wl1, bl1, Wl2, bl2):
    return pl.pallas_call(
        _final_body,
        grid=(NBLK,),
        in_specs=[
            pl.BlockSpec((BR, HALF), lambda i: (i, 0)),
            pl.BlockSpec((BR, HALF), lambda i: (i, 0)),
            pl.BlockSpec((D_H, D_H), lambda i: (0, 0)),
            pl.BlockSpec((D_H,), lambda i: (0,)),
            pl.BlockSpec((D_H, D_H), lambda i: (0, 0)),
            pl.BlockSpec((D_H,), lambda i: (0,)),
            pl.BlockSpec((1, 1, BR), lambda i: (i, 0, 0)),
            pl.BlockSpec((D_H, D_H), lambda i: (0, 0)),
            pl.BlockSpec((D_H,), lambda i: (0,)),
            pl.BlockSpec((D_H, N_CLASSES), lambda i: (0, 0)),
            pl.BlockSpec((N_CLASSES,), lambda i: (0,)),
        ],
        out_specs=pl.BlockSpec((N_GRAPHS, N_CLASSES), lambda i: (0, 0)),
        out_shape=jax.ShapeDtypeStruct((N_GRAPHS, N_CLASSES), jnp.float32),
        scratch_shapes=[pltpu.VMEM((N_GRAPHS, D_H), jnp.float32)],
    )(za, zb, W1, b1, W2, b2, batch3, Wl1, bl1, Wl2, bl2)


def kernel(x, edge_index, batch, We1, be1, We2, be2, Wc0a, bc0a, Wc0b, bc0b,
           Wc1a, bc1a, Wc1b, bc1b, Wl1, bl1, Wl2, bl2):
    src = edge_index[0].astype(jnp.int32)
    dst = edge_index[1].astype(jnp.int32)
    batch3 = batch.astype(jnp.int32).reshape(NBLK, 1, BR)

    hA, hB = _enc(x, We1, be1, We2, be2)
    zA, zB = _sc_segsum(hA, hB, src, dst)
    h2A, h2B = _mlp(zA, zB, Wc0a, bc0a, Wc0b, bc0b)
    z2A, z2B = _sc_segsum(h2A, h2B, src, dst)
    return _final(z2A, z2B, Wc1a, bc1a, Wc1b, bc1b, batch3, Wl1, bl1, Wl2, bl2)


# speedup vs baseline: 11.4129x; 1.0026x over previous
import functools

import jax
import jax.numpy as jnp
from jax import lax
from jax.experimental import pallas as pl
from jax.experimental.pallas import tpu as pltpu
from jax.experimental.pallas import tpu_sc as plsc

N = 10000
E = 320000
D_IN = 128
D_H = 256
HALF = 128
N_CLASSES = 10
N_GRAPHS = 64

NC = 2
NS = 16
EPW = E // NS
K = 40
NCHUNK = EPW // K
NBUF = 5
GD = 4
NI = 10
IDL = 8
RPT = 632
RPT_LAST = N - (NS - 1) * RPT

BR = 2000
NBLK = N // BR


def _rowwise_copy(c, s, srcs, dsts):
    base = pl.multiple_of(s * RPT, 8)
    for cid in (0, 1):
        for sz, b in ((RPT, base), (RPT_LAST, (NS - 1) * RPT)):
            is_last = sz == RPT_LAST

            @pl.when((c == cid) & ((s == NS - 1) == is_last))
            def _(cid=cid, sz=sz, b=b):
                pltpu.sync_copy(srcs[cid].at[pl.ds(b, sz)],
                                dsts[cid].at[pl.ds(b, sz)])


def _sc_segsum_body(hA, hB, src, dst, zA, zB, acc, *bufs):
    rows = bufs[:NBUF]
    sidx = bufs[NBUF:NBUF + NI]
    didx = bufs[NBUF + NI]
    gsem = bufs[NBUF + NI + 1:2 * NBUF + NI + 1]
    ssem = bufs[2 * NBUF + NI + 1:3 * NBUF + NI + 1]
    isem = bufs[3 * NBUF + NI + 1:3 * NBUF + 2 * NI + 1]
    dsem = bufs[3 * NBUF + 2 * NI + 1]
    c = lax.axis_index("c")
    s = lax.axis_index("s")

    def i_descs(jc, bi):
        off = pl.multiple_of(s * EPW + jc * K, 8)
        return (pltpu.make_async_copy(src.at[pl.ds(off, K)], sidx[bi],
                                      isem[bi]),)

    def g_desc(table, jc, b, bi):
        return pltpu.make_async_copy(table.at[sidx[bi]], rows[b], gsem[b])

    def s_desc(jc, b, bi):
        off = pl.multiple_of(jc * K, 8)
        return pltpu.make_async_copy(rows[b], acc.at[didx.at[pl.ds(off, K)]],
                                     ssem[b])

    def issue_gather(jc, b, bi):
        for d in i_descs(jc, bi):
            d.wait()

        @pl.when(c == 0)
        def _():
            g_desc(hA, jc, b, bi).start()

        @pl.when(c == 1)
        def _():
            g_desc(hB, jc, b, bi).start()

    ebase = pl.multiple_of(s * EPW, 8)
    dcp = pltpu.async_copy(dst.at[pl.ds(ebase, EPW)], didx, dsem)
    for bi in range(IDL):
        for d in i_descs(bi, bi):
            d.start()
    _rowwise_copy(c, s, (hA, hB), (acc, acc))
    dcp.wait()
    for b in range(GD):
        issue_gather(b, b, b)

    plsc.subcore_barrier()

    @pl.loop(0, NCHUNK, step=NI)
    def _(j):
        for b0 in range(NI):
            jc = j + b0
            b = b0 % NBUF
            b2 = (b0 + GD) % NBUF
            bi2 = (b0 + GD) % NI
            bi3 = (b0 + IDL) % NI

            @pl.when(jc + GD < NCHUNK)
            def _():
                @pl.when(jc + GD >= NBUF)
                def _():
                    s_desc(jc + GD - NBUF, b2, (b0 + GD - NBUF) % NI).wait()

                issue_gather(jc + GD, b2, bi2)

            g_desc(hA, jc, b, b0).wait()
            s_desc(jc, b, b0).start(add=True)

            @pl.when(jc + IDL < NCHUNK)
            def _():
                for d in i_descs(jc + IDL, bi3):
                    d.start()

    for b0 in range(NBUF):
        jc_last = NCHUNK - NBUF + b0
        s_desc(jc_last, jc_last % NBUF, jc_last % NI).wait()

    plsc.subcore_barrier()

    _rowwise_copy(c, s, (acc, acc), (zA, zB))


@functools.cache
def _sc_segsum_call():
    return pl.kernel(
        _sc_segsum_body,
        out_type=(
            jax.ShapeDtypeStruct((N, HALF), jnp.float32),
            jax.ShapeDtypeStruct((N, HALF), jnp.float32),
        ),
        mesh=plsc.VectorSubcoreMesh(core_axis_name="c", subcore_axis_name="s"),
        scratch_types=(
            [pltpu.VMEM_SHARED((N, HALF), jnp.float32)]
            + [pltpu.VMEM((K, HALF), jnp.float32) for _ in range(NBUF)]
            + [pltpu.VMEM((K,), jnp.int32) for _ in range(NI)]
            + [pltpu.VMEM((EPW,), jnp.int32)]
            + [pltpu.SemaphoreType.DMA for _ in range(2 * NBUF + NI + 1)]
        ),
    )


def _sc_segsum(hA, hB, src, dst):
    return _sc_segsum_call()(hA, hB, src, dst)


def _enc_body(x_ref, w1, b1, w2, b2, oa, ob):
    h = jnp.maximum(jnp.dot(x_ref[...], w1[...],
                            preferred_element_type=jnp.float32) + b1[...], 0.0)
    h = jnp.maximum(jnp.dot(h, w2[...],
                            preferred_element_type=jnp.float32) + b2[...], 0.0)
    oa[...] = h[:, :HALF]
    ob[...] = h[:, HALF:]


def _enc(x, W1, b1, W2, b2):
    return pl.pallas_call(
        _enc_body,
        grid=(NBLK,),
        in_specs=[
            pl.BlockSpec((BR, D_IN), lambda i: (i, 0)),
            pl.BlockSpec((D_IN, D_H), lambda i: (0, 0)),
            pl.BlockSpec((D_H,), lambda i: (0,)),
            pl.BlockSpec((D_H, D_H), lambda i: (0, 0)),
            pl.BlockSpec((D_H,), lambda i: (0,)),
        ],
        out_specs=[
            pl.BlockSpec((BR, HALF), lambda i: (i, 0)),
            pl.BlockSpec((BR, HALF), lambda i: (i, 0)),
        ],
        out_shape=[
            jax.ShapeDtypeStruct((N, HALF), jnp.float32),
            jax.ShapeDtypeStruct((N, HALF), jnp.float32),
        ],
    )(x, W1, b1, W2, b2)


def _mlp_body(za, zb, w1, b1, w2, b2, oa, ob):
    w1v = w1[...]
    t = (jnp.dot(za[...], w1v[:HALF], preferred_element_type=jnp.float32)
         + jnp.dot(zb[...], w1v[HALF:], preferred_element_type=jnp.float32)
         + b1[...])
    t = jnp.maximum(t, 0.0)
    o = jnp.dot(t, w2[...], preferred_element_type=jnp.float32) + b2[...]
    oa[...] = o[:, :HALF]
    ob[...] = o[:, HALF:]


def _mlp(za, zb, W1, b1, W2, b2):
    return pl.pallas_call(
        _mlp_body,
        grid=(NBLK,),
        in_specs=[
            pl.BlockSpec((BR, HALF), lambda i: (i, 0)),
            pl.BlockSpec((BR, HALF), lambda i: (i, 0)),
            pl.BlockSpec((D_H, D_H), lambda i: (0, 0)),
            pl.BlockSpec((D_H,), lambda i: (0,)),
            pl.BlockSpec((D_H, D_H), lambda i: (0, 0)),
            pl.BlockSpec((D_H,), lambda i: (0,)),
        ],
        out_specs=[
            pl.BlockSpec((BR, HALF), lambda i: (i, 0)),
            pl.BlockSpec((BR, HALF), lambda i: (i, 0)),
        ],
        out_shape=[
            jax.ShapeDtypeStruct((N, HALF), jnp.float32),
            jax.ShapeDtypeStruct((N, HALF), jnp.float32),
        ],
    )(za, zb, W1, b1, W2, b2)


def _final_body(za, zb, w1, b1, w2, b2, batch_ref, wl1, bl1, wl2, bl2,
                out_ref, pooled):
    i = pl.program_id(0)
    w1v = w1[...]
    t = (jnp.dot(za[...], w1v[:HALF], preferred_element_type=jnp.float32)
         + jnp.dot(zb[...], w1v[HALF:], preferred_element_type=jnp.float32)
         + b1[...])
    t = jnp.maximum(t, 0.0)
    h3 = jnp.dot(t, w2[...], preferred_element_type=jnp.float32) + b2[...]

    b_ids = batch_ref[0, 0, :]
    gids = lax.broadcasted_iota(jnp.int32, (N_GRAPHS, BR), 0)
    onehot = (b_ids[None, :] == gids).astype(jnp.float32)
    part = jnp.dot(onehot, h3, preferred_element_type=jnp.float32)

    @pl.when(i == 0)
    def _():
        pooled[...] = part

    @pl.when(i > 0)
    def _():
        pooled[...] = pooled[...] + part

    @pl.when(i == NBLK - 1)
    def _():
        p = jnp.maximum(jnp.dot(pooled[...], wl1[...],
                                preferred_element_type=jnp.float32) + bl1[...],
                        0.0)
        out_ref[...] = jnp.dot(p, wl2[...],
                               preferred_element_type=jnp.float32) + bl2[...]


def _final(za, zb, W1, b1, W2, b2, batch3, Wl1, bl1, Wl2, bl2):
    return pl.pallas_call(
        _final_body,
        grid=(NBLK,),
        in_specs=[
            pl.BlockSpec((BR, HALF), lambda i: (i, 0)),
            pl.BlockSpec((BR, HALF), lambda i: (i, 0)),
            pl.BlockSpec((D_H, D_H), lambda i: (0, 0)),
            pl.BlockSpec((D_H,), lambda i: (0,)),
            pl.BlockSpec((D_H, D_H), lambda i: (0, 0)),
            pl.BlockSpec((D_H,), lambda i: (0,)),
            pl.BlockSpec((1, 1, BR), lambda i: (i, 0, 0)),
            pl.BlockSpec((D_H, D_H), lambda i: (0, 0)),
            pl.BlockSpec((D_H,), lambda i: (0,)),
            pl.BlockSpec((D_H, N_CLASSES), lambda i: (0, 0)),
            pl.BlockSpec((N_CLASSES,), lambda i: (0,)),
        ],
        out_specs=pl.BlockSpec((N_GRAPHS, N_CLASSES), lambda i: (0, 0)),
        out_shape=jax.ShapeDtypeStruct((N_GRAPHS, N_CLASSES), jnp.float32),
        scratch_shapes=[pltpu.VMEM((N_GRAPHS, D_H), jnp.float32)],
    )(za, zb, W1, b1, W2, b2, batch3, Wl1, bl1, Wl2, bl2)


def kernel(x, edge_index, batch, We1, be1, We2, be2, Wc0a, bc0a, Wc0b, bc0b,
           Wc1a, bc1a, Wc1b, bc1b, Wl1, bl1, Wl2, bl2):
    src = edge_index[0].astype(jnp.int32)
    dst = edge_index[1].astype(jnp.int32)
    batch3 = batch.astype(jnp.int32).reshape(NBLK, 1, BR)

    hA, hB = _enc(x, We1, be1, We2, be2)
    zA, zB = _sc_segsum(hA, hB, src, dst)
    h2A, h2B = _mlp(zA, zB, Wc0a, bc0a, Wc0b, bc0b)
    z2A, z2B = _sc_segsum(h2A, h2B, src, dst)
    return _final(z2A, z2B, Wc1a, bc1a, Wc1b, bc1b, batch3, Wl1, bl1, Wl2, bl2)
